# Initial kernel scaffold; baseline (speedup 1.0000x reference)
import jax
import jax.numpy as jnp
from jax.experimental import pallas as pl


def kernel(inputs, vt_replace, vt_map, vt_out):
    raise NotImplementedError("write your pallas kernel here")


# speedup vs baseline: 1.5193x; 1.5193x over previous
import jax
import jax.numpy as jnp
from jax import lax
from jax.experimental import pallas as pl
from jax.experimental.pallas import tpu as pltpu
from jax.experimental.pallas import tpu_sc as plsc

N_IN = 100000
N_OUT = 50000
D = 128
L = 16
NC = 2
NS = 16
NW = NC * NS
T = 400
NT = N_OUT // T
TPW = (NT + NW - 1) // NW
C = 256
NEG = -(2**31) + 1


def _worker(in_hbm, vtm_hbm, bnd_hbm, out_hbm, in_buf, vtm_buf, bnd_buf, stg):
    wid = lax.axis_index("c") * NS + lax.axis_index("s")
    ninf = jnp.full((L,), -jnp.inf, jnp.float32)
    zero = jnp.zeros((L,), jnp.float32)

    for tslot in range(TPW):
        tile_id = wid + tslot * NW

        @pl.when(tile_id < NT)
        def _process():
            tile_lo = tile_id * T
            pltpu.sync_copy(bnd_hbm.at[tile_id], bnd_buf)
            bv = bnd_buf[...]
            i_start = bv[0]
            i_end = bv[1]

            def zrow(r, carry):
                for f in range(D // L):
                    stg[r, pl.ds(f * L, L)] = ninf
                return carry

            lax.fori_loop(0, T, zrow, 0)

            row0 = jnp.minimum((i_start // 8) * 8, N_IN - C)
            nch = (i_end - row0 + (C - 1)) // C

            def chunk(cidx, carry):
                r0 = jnp.minimum(row0 + cidx * C, N_IN - C)
                pltpu.sync_copy(vtm_hbm.at[pl.ds(r0, C)], vtm_buf)
                pltpu.sync_copy(in_hbm.at[pl.ds(r0, C)], in_buf)

                def group(g, gcarry):
                    pos = vtm_buf[pl.ds(g * L, L)] - tile_lo
                    for r in range(L):
                        row = g * L + r
                        p = pos[r]

                        @pl.when((p >= 0) & (p < T))
                        def _rmw():
                            for f in range(D // L):
                                x = in_buf[row, pl.ds(f * L, L)]
                                cur = stg[p, pl.ds(f * L, L)]
                                stg[p, pl.ds(f * L, L)] = jnp.maximum(cur, x)

                    return gcarry

                lax.fori_loop(0, C // L, group, 0)
                return carry

            lax.fori_loop(0, nch, chunk, 0)

            def frow(r, carry):
                for f in range(D // L):
                    v = stg[r, pl.ds(f * L, L)]
                    stg[r, pl.ds(f * L, L)] = jnp.where(v == -jnp.inf, zero, v)
                return carry

            lax.fori_loop(0, T, frow, 0)
            pltpu.sync_copy(stg, out_hbm.at[pl.ds(tile_lo, T)])


def kernel(inputs, vt_replace, vt_map, vt_out):
    del vt_replace, vt_out
    vtm = jnp.clip(vt_map.astype(jnp.int32), 0, N_OUT - 1)
    edges = jnp.arange(NT + 1, dtype=jnp.int32) * T
    b = jnp.searchsorted(vtm, edges, side="left").astype(jnp.int32)
    bnd = jnp.zeros((NT, L), dtype=jnp.int32)
    bnd = bnd.at[:, 0].set(b[:-1])
    bnd = bnd.at[:, 1].set(b[1:])

    mesh = plsc.VectorSubcoreMesh(core_axis_name="c", subcore_axis_name="s")
    f = pl.kernel(
        _worker,
        out_type=jax.ShapeDtypeStruct((N_OUT, D), jnp.float32),
        mesh=mesh,
        scratch_types=[
            pltpu.VMEM((C, D), jnp.float32),
            pltpu.VMEM((C,), jnp.int32),
            pltpu.VMEM((L,), jnp.int32),
            pltpu.VMEM((T, D), jnp.float32),
        ],
    )
    return f(inputs, vtm, bnd)


# speedup vs baseline: 2.5648x; 1.6881x over previous
import jax
import jax.numpy as jnp
from jax import lax
from jax.experimental import pallas as pl
from jax.experimental.pallas import tpu as pltpu
from jax.experimental.pallas import tpu_sc as plsc

N_IN = 100000
N_OUT = 50000
D = 128
L = 16
NF = D // L
NC = 2
NS = 16
NW = NC * NS
T = 400
NT = N_OUT // T
TPW = (NT + NW - 1) // NW
C = 160


def _worker(in_hbm, vtm_hbm, bnd_hbm, out_hbm, in_buf, vtm_buf, bnd_buf, stg):
    wid = lax.axis_index("c") * NS + lax.axis_index("s")
    zero = jnp.zeros((L,), jnp.float32)
    ninf = jnp.full((L,), -jnp.inf, jnp.float32)

    for tslot in range(TPW):
        tile_id = wid + tslot * NW

        @pl.when(tile_id < NT)
        def _process():
            tile_lo = tile_id * T
            pltpu.sync_copy(bnd_hbm.at[tile_id], bnd_buf)
            bv = bnd_buf[...]
            i_start = bv[0]
            i_end = bv[1]

            def zrow(r, carry):
                for f in range(NF):
                    stg[r, pl.ds(f * L, L)] = zero
                return carry

            lax.fori_loop(0, T, zrow, 0)

            k0 = i_start // C
            nch = jnp.where(i_end > i_start, (i_end + C - 1) // C - k0, 0)

            def chunk(cidx, carry):
                r0 = (k0 + cidx) * C
                pltpu.sync_copy(vtm_hbm.at[pl.ds(r0, C)], vtm_buf)
                pltpu.sync_copy(in_hbm.at[pl.ds(r0, C)], in_buf)

                def group(g, gc):
                    ck, cp, accs = gc
                    posv = vtm_buf[pl.ds(g * L, L)] - tile_lo
                    for r in range(L):
                        p = posv[r]
                        valid = (p >= 0) & (p < T)
                        pc = jnp.where(valid, p, T)
                        key = jnp.where(valid, p, -1)
                        same = key == ck
                        gate = jnp.where(same, jnp.float32(0), -jnp.inf)
                        gate_v = jnp.full((L,), 1.0, jnp.float32) * gate
                        row = g * L + r
                        new_accs = []
                        for f in range(NF):
                            x = in_buf[row, pl.ds(f * L, L)]
                            stg[cp, pl.ds(f * L, L)] = accs[f]
                            new_accs.append(
                                jnp.maximum(accs[f] + gate_v, x))
                        accs = tuple(new_accs)
                        ck, cp = key, pc
                    return (ck, cp, accs)

                return lax.fori_loop(0, C // L, group, carry)

            init_accs = tuple(
                stg[0, pl.ds(f * L, L)] for f in range(NF))
            init = (jnp.int32(-2), jnp.int32(T), init_accs)
            ck, cp, accs = lax.fori_loop(0, nch, chunk, init)
            for f in range(NF):
                stg[cp, pl.ds(f * L, L)] = accs[f]
            pltpu.sync_copy(stg.at[pl.ds(0, T)], out_hbm.at[pl.ds(tile_lo, T)])


def kernel(inputs, vt_replace, vt_map, vt_out):
    del vt_replace, vt_out
    vtm = jnp.clip(vt_map.astype(jnp.int32), 0, N_OUT - 1)
    edges = jnp.arange(NT + 1, dtype=jnp.int32) * T
    b = jnp.searchsorted(vtm, edges, side="left").astype(jnp.int32)
    bnd = jnp.zeros((NT, L), dtype=jnp.int32)
    bnd = bnd.at[:, 0].set(b[:-1])
    bnd = bnd.at[:, 1].set(b[1:])

    mesh = plsc.VectorSubcoreMesh(core_axis_name="c", subcore_axis_name="s")
    f = pl.kernel(
        _worker,
        out_type=jax.ShapeDtypeStruct((N_OUT, D), jnp.float32),
        mesh=mesh,
        scratch_types=[
            pltpu.VMEM((C, D), jnp.float32),
            pltpu.VMEM((C,), jnp.int32),
            pltpu.VMEM((L,), jnp.int32),
            pltpu.VMEM((T + 8, D), jnp.float32),
        ],
    )
    return f(inputs, vtm, bnd)


# speedup vs baseline: 3.5883x; 1.3990x over previous
import jax
import jax.numpy as jnp
from jax import lax
from jax.experimental import pallas as pl
from jax.experimental.pallas import tpu as pltpu
from jax.experimental.pallas import tpu_sc as plsc

N_IN = 100000
N_OUT = 50000
D = 128
L = 16
NF = D // L
NC = 2
NS = 16
NW = NC * NS
T = 400
NT = N_OUT // T
TPW = (NT + NW - 1) // NW
C = 160


def _worker(in_hbm, vtm_hbm, bnd_hbm, out_hbm,
            in_a, in_b, vtm_a, vtm_b, bnd_buf, stg, sem_a, sem_b):
    wid = lax.axis_index("c") * NS + lax.axis_index("s")
    zero = jnp.zeros((L,), jnp.float32)

    for tslot in range(TPW):
        tile_id = wid + tslot * NW

        @pl.when(tile_id < NT)
        def _process_tile():
            tile_lo = tile_id * T
            pltpu.sync_copy(bnd_hbm.at[tile_id], bnd_buf)
            bv = bnd_buf[...]
            i_start = bv[0]
            i_end = bv[1]

            k0 = i_start // C
            nch = jnp.where(i_end > i_start, (i_end + C - 1) // C - k0, 0)

            def start(c, inb, vtb, sem):
                @pl.when(c < nch)
                def _():
                    r0 = (k0 + c) * C
                    pltpu.async_copy(vtm_hbm.at[pl.ds(r0, C)], vtb, sem)
                    pltpu.async_copy(in_hbm.at[pl.ds(r0, C)], inb, sem)

            def wait(c, inb, vtb, sem):
                @pl.when(c < nch)
                def _():
                    pltpu.make_async_copy(
                        vtm_hbm.at[pl.ds(0, C)], vtb, sem).wait()
                    pltpu.make_async_copy(
                        in_hbm.at[pl.ds(0, C)], inb, sem).wait()

            start(0, in_a, vtm_a, sem_a)
            start(1, in_b, vtm_b, sem_b)

            def zrow(r, carry):
                for f in range(NF):
                    stg[r, pl.ds(f * L, L)] = zero
                return carry

            lax.fori_loop(0, T, zrow, 0)

            def process(inb, vtb, ok, carry):
                def group(g, gc):
                    ck, cp, accs = gc
                    posv = vtb[pl.ds(g * L, L)] - tile_lo
                    for r in range(L):
                        p = posv[r]
                        vok = ok & (p >= 0) & (p < T)
                        pc = jnp.where(vok, p, T)
                        key = jnp.where(vok, p, -1)
                        same = key == ck
                        gate = jnp.where(same, jnp.float32(0), -jnp.inf)
                        gate_v = jnp.full((L,), 1.0, jnp.float32) * gate
                        row = g * L + r
                        new_accs = []
                        for f in range(NF):
                            x = inb[row, pl.ds(f * L, L)]
                            stg[cp, pl.ds(f * L, L)] = accs[f]
                            new_accs.append(
                                jnp.maximum(accs[f] + gate_v, x))
                        accs = tuple(new_accs)
                        ck, cp = key, pc
                    return (ck, cp, accs)

                return lax.fori_loop(0, C // L, group, carry)

            def pair(pidx, carry):
                c0 = 2 * pidx
                wait(c0, in_a, vtm_a, sem_a)
                carry = process(in_a, vtm_a, c0 < nch, carry)
                start(c0 + 2, in_a, vtm_a, sem_a)
                c1 = c0 + 1
                wait(c1, in_b, vtm_b, sem_b)
                carry = process(in_b, vtm_b, c1 < nch, carry)
                start(c1 + 2, in_b, vtm_b, sem_b)
                return carry

            init_accs = tuple(
                stg[0, pl.ds(f * L, L)] for f in range(NF))
            init = (jnp.int32(-2), jnp.int32(T), init_accs)
            ck, cp, accs = lax.fori_loop(0, (nch + 1) // 2, pair, init)
            for f in range(NF):
                stg[cp, pl.ds(f * L, L)] = accs[f]
            pltpu.sync_copy(stg.at[pl.ds(0, T)], out_hbm.at[pl.ds(tile_lo, T)])


def kernel(inputs, vt_replace, vt_map, vt_out):
    del vt_replace, vt_out
    vtm = jnp.clip(vt_map.astype(jnp.int32), 0, N_OUT - 1)
    edges = jnp.arange(NT + 1, dtype=jnp.int32) * T
    b = jnp.searchsorted(vtm, edges, side="left").astype(jnp.int32)
    bnd = jnp.zeros((NT, L), dtype=jnp.int32)
    bnd = bnd.at[:, 0].set(b[:-1])
    bnd = bnd.at[:, 1].set(b[1:])

    mesh = plsc.VectorSubcoreMesh(core_axis_name="c", subcore_axis_name="s")
    f = pl.kernel(
        _worker,
        out_type=jax.ShapeDtypeStruct((N_OUT, D), jnp.float32),
        mesh=mesh,
        scratch_types=[
            pltpu.VMEM((C, D), jnp.float32),
            pltpu.VMEM((C, D), jnp.float32),
            pltpu.VMEM((C,), jnp.int32),
            pltpu.VMEM((C,), jnp.int32),
            pltpu.VMEM((L,), jnp.int32),
            pltpu.VMEM((T + 8, D), jnp.float32),
            pltpu.SemaphoreType.DMA,
            pltpu.SemaphoreType.DMA,
        ],
    )
    return f(inputs, vtm, bnd)


# speedup vs baseline: 4.0589x; 1.1312x over previous
import jax
import jax.numpy as jnp
from jax import lax
from jax.experimental import pallas as pl
from jax.experimental.pallas import tpu as pltpu
from jax.experimental.pallas import tpu_sc as plsc

N_IN = 100000
N_OUT = 50000
D = 128
L = 16
NF = D // L
NC = 2
NS = 16
NW = NC * NS
T = 400
NT = N_OUT // T
TPW = (NT + NW - 1) // NW
C = 160


def _worker(in_hbm, vtm_hbm, bnd_hbm, out_hbm,
            in_a, in_b, vtm_a, vtm_b, bnd_buf, stg, sem_a, sem_b):
    wid = lax.axis_index("c") * NS + lax.axis_index("s")
    zero = jnp.zeros((L,), jnp.float32)

    for tslot in range(TPW):
        tile_id = wid + tslot * NW

        @pl.when(tile_id < NT)
        def _process_tile():
            tile_lo = tile_id * T
            pltpu.sync_copy(bnd_hbm.at[tile_id], bnd_buf)
            bv = bnd_buf[...]
            i_start = bv[0]
            i_end = bv[1]

            k0 = i_start // C
            nch = jnp.where(i_end > i_start, (i_end + C - 1) // C - k0, 0)

            def start(c, inb, vtb, sem):
                @pl.when(c < nch)
                def _():
                    r0 = (k0 + c) * C
                    pltpu.async_copy(
                        vtm_hbm.at[pl.ds(r0, C)], vtb.at[pl.ds(L, C)], sem)
                    pltpu.async_copy(in_hbm.at[pl.ds(r0, C)], inb, sem)

            def wait(c, inb, vtb, sem):
                @pl.when(c < nch)
                def _():
                    pltpu.make_async_copy(
                        vtm_hbm.at[pl.ds(0, C)], vtb.at[pl.ds(L, C)],
                        sem).wait()
                    pltpu.make_async_copy(
                        in_hbm.at[pl.ds(0, C)], inb, sem).wait()

            start(0, in_a, vtm_a, sem_a)
            start(1, in_b, vtm_b, sem_b)

            def zrow(r, carry):
                for f in range(NF):
                    stg[r, pl.ds(f * L, L)] = zero
                return carry

            lax.fori_loop(0, T, zrow, 0)

            def process(inb, vtb, ok, carry):
                ng = jnp.where(ok, C // L, 0)

                def group(g, gc):
                    cp, accs = gc
                    segv = vtb[pl.ds(L + g * L, L)]
                    prevv = vtb[pl.ds(L - 1 + g * L, L)]
                    posv = segv - tile_lo
                    validv = (posv >= 0) & (posv < T)
                    pcv = jnp.where(validv, posv, T)
                    gatev = jnp.where(segv == prevv,
                                      jnp.float32(0), -jnp.inf)
                    for r in range(L):
                        pc = pcv[r]
                        gate = gatev[r]
                        row = g * L + r
                        new_accs = []
                        for f in range(NF):
                            x = inb[row, pl.ds(f * L, L)]
                            stg[cp, pl.ds(f * L, L)] = accs[f]
                            new_accs.append(
                                jnp.maximum(accs[f] + gate, x))
                        accs = tuple(new_accs)
                        cp = pc
                    return (cp, accs)

                return lax.fori_loop(0, ng, group, carry)

            def copy_tail(src_vtb, dst_vtb):
                dst_vtb[pl.ds(0, L)] = src_vtb[pl.ds(C, L)]

            def pair(pidx, carry):
                c0 = 2 * pidx
                wait(c0, in_a, vtm_a, sem_a)
                carry = process(in_a, vtm_a, c0 < nch, carry)
                copy_tail(vtm_a, vtm_b)
                start(c0 + 2, in_a, vtm_a, sem_a)
                c1 = c0 + 1
                wait(c1, in_b, vtm_b, sem_b)
                carry = process(in_b, vtm_b, c1 < nch, carry)
                copy_tail(vtm_b, vtm_a)
                start(c1 + 2, in_b, vtm_b, sem_b)
                return carry

            vtm_a[pl.ds(0, L)] = jnp.full((L,), -1, jnp.int32)
            init_accs = tuple(
                stg[0, pl.ds(f * L, L)] for f in range(NF))
            init = (jnp.int32(T), init_accs)
            cp, accs = lax.fori_loop(0, (nch + 1) // 2, pair, init)
            for f in range(NF):
                stg[cp, pl.ds(f * L, L)] = accs[f]
            pltpu.sync_copy(stg.at[pl.ds(0, T)], out_hbm.at[pl.ds(tile_lo, T)])


def kernel(inputs, vt_replace, vt_map, vt_out):
    del vt_replace, vt_out
    vtm = jnp.clip(vt_map.astype(jnp.int32), 0, N_OUT - 1)
    edges = jnp.arange(NT + 1, dtype=jnp.int32) * T
    b = jnp.searchsorted(vtm, edges, side="left").astype(jnp.int32)
    bnd = jnp.zeros((NT, L), dtype=jnp.int32)
    bnd = bnd.at[:, 0].set(b[:-1])
    bnd = bnd.at[:, 1].set(b[1:])

    mesh = plsc.VectorSubcoreMesh(core_axis_name="c", subcore_axis_name="s")
    f = pl.kernel(
        _worker,
        out_type=jax.ShapeDtypeStruct((N_OUT, D), jnp.float32),
        mesh=mesh,
        scratch_types=[
            pltpu.VMEM((C, D), jnp.float32),
            pltpu.VMEM((C, D), jnp.float32),
            pltpu.VMEM((C + L,), jnp.int32),
            pltpu.VMEM((C + L,), jnp.int32),
            pltpu.VMEM((L,), jnp.int32),
            pltpu.VMEM((T + 8, D), jnp.float32),
            pltpu.SemaphoreType.DMA,
            pltpu.SemaphoreType.DMA,
        ],
    )
    return f(inputs, vtm, bnd)
